# baseline (speedup 1.0000x reference)
import jax
import jax.numpy as jnp
from jax import lax
from jax.experimental import pallas as pl
from jax.experimental.pallas import tpu as pltpu
from jax.experimental.pallas import tpu_sc as plsc

H = 32
W = 32
_LANES = 16
_SC_WORKERS = 32


def _pos_sc_kernel(row_hbm, col_hbm, r_hbm, c_hbm, pos_hbm,
                   idx_r, idx_c, rows_r, rows_c, sem_r, sem_c):
    chunk = rows_r.shape[0]
    d = rows_r.shape[1]
    wid = lax.axis_index("s") * 2 + lax.axis_index("c")
    base = wid * chunk
    pltpu.sync_copy(r_hbm.at[pl.ds(base, chunk)], idx_r)
    pltpu.sync_copy(c_hbm.at[pl.ds(base, chunk)], idx_c)
    cp_r = pltpu.async_copy(row_hbm.at[idx_r], rows_r, sem_r)
    cp_c = pltpu.async_copy(col_hbm.at[idx_c], rows_c, sem_c)
    cp_r.wait()
    cp_c.wait()

    def body(i, carry):
        for j in range(d // _LANES):
            sl = pl.ds(j * _LANES, _LANES)
            rows_r[i, sl] = rows_r[i, sl] + rows_c[i, sl]
        return carry

    lax.fori_loop(0, chunk, body, 0)
    pltpu.sync_copy(rows_r, pos_hbm.at[pl.ds(base, chunk)])


def _gather_pos_sc(row_table, col_table, r, c):
    S = r.shape[0]
    D = row_table.shape[1]
    chunk = S // _SC_WORKERS
    return pl.kernel(
        _pos_sc_kernel,
        out_type=jax.ShapeDtypeStruct((S, D), jnp.float32),
        mesh=plsc.VectorSubcoreMesh(core_axis_name="c", subcore_axis_name="s"),
        scratch_types=[
            pltpu.VMEM((chunk,), jnp.int32),
            pltpu.VMEM((chunk,), jnp.int32),
            pltpu.VMEM((chunk, D), jnp.float32),
            pltpu.VMEM((chunk, D), jnp.float32),
            pltpu.SemaphoreType.DMA,
            pltpu.SemaphoreType.DMA,
        ],
    )(row_table, col_table, r, c)


def _add_pos_kernel(pos_ref, out_in_ref, out_ref):
    s = pos_ref.shape[0]
    nrep = out_ref.shape[0] // s
    for i in range(nrep):
        out_ref[i * s:(i + 1) * s, :] = (
            out_in_ref[i * s:(i + 1) * s, :] + pos_ref[...])


_BATCHES_PER_BLOCK = 8


def kernel(output, row_table, col_table, r, c):
    B, S, D = output.shape
    pos = _gather_pos_sc(row_table, col_table, r, c)
    flat = output.reshape(B * S, D)
    nb = _BATCHES_PER_BLOCK
    rows = nb * S
    res = pl.pallas_call(
        _add_pos_kernel,
        grid=(B // nb,),
        in_specs=[
            pl.BlockSpec((S, D), lambda b: (0, 0)),
            pl.BlockSpec((rows, D), lambda b: (b, 0)),
        ],
        out_specs=pl.BlockSpec((rows, D), lambda b: (b, 0)),
        out_shape=jax.ShapeDtypeStruct((B * S, D), jnp.float32),
    )(pos, flat)
    return res.reshape(B, S, D)


# speedup vs baseline: 1.0470x; 1.0470x over previous
import jax
import jax.numpy as jnp
from jax import lax
from jax.experimental import pallas as pl
from jax.experimental.pallas import tpu as pltpu
from jax.experimental.pallas import tpu_sc as plsc

H = 32
W = 32
_LANES = 16
_SC_WORKERS = 32

_BATCHES_PER_BLOCK = 8
_TC1_BATCHES = 24


def _pos_sc_kernel(row_hbm, col_hbm, r_hbm, c_hbm, pos_hbm,
                   idx_r, idx_c, rows_r, rows_c, sem_r, sem_c):
    chunk = rows_r.shape[0]
    d = rows_r.shape[1]
    wid = lax.axis_index("s") * 2 + lax.axis_index("c")
    base = wid * chunk
    pltpu.sync_copy(r_hbm.at[pl.ds(base, chunk)], idx_r)
    pltpu.sync_copy(c_hbm.at[pl.ds(base, chunk)], idx_c)
    cp_r = pltpu.async_copy(row_hbm.at[idx_r], rows_r, sem_r)
    cp_c = pltpu.async_copy(col_hbm.at[idx_c], rows_c, sem_c)
    cp_r.wait()
    cp_c.wait()

    def body(i, carry):
        for j in range(d // _LANES):
            sl = pl.ds(j * _LANES, _LANES)
            rows_r[i, sl] = rows_r[i, sl] + rows_c[i, sl]
        return carry

    lax.fori_loop(0, chunk, body, 0)
    pltpu.sync_copy(rows_r, pos_hbm.at[pl.ds(base, chunk)])


def _gather_pos_sc(row_table, col_table, r, c):
    S = r.shape[0]
    D = row_table.shape[1]
    chunk = S // _SC_WORKERS
    return pl.kernel(
        _pos_sc_kernel,
        out_type=jax.ShapeDtypeStruct((S, D), jnp.float32),
        mesh=plsc.VectorSubcoreMesh(core_axis_name="c", subcore_axis_name="s"),
        scratch_types=[
            pltpu.VMEM((chunk,), jnp.int32),
            pltpu.VMEM((chunk,), jnp.int32),
            pltpu.VMEM((chunk, D), jnp.float32),
            pltpu.VMEM((chunk, D), jnp.float32),
            pltpu.SemaphoreType.DMA,
            pltpu.SemaphoreType.DMA,
        ],
    )(row_table, col_table, r, c)


def _tc1_kernel(r_ref, c_ref, row_tab_ref, col_tab_ref, out_in_ref,
                out_ref, pos_ref):
    b = pl.program_id(0)

    @pl.when(b == 0)
    def _():
        s = r_ref.shape[0]
        row_oh = (jax.lax.broadcasted_iota(jnp.int32, (s, H), 1)
                  == r_ref[...]).astype(jnp.float32)
        col_oh = (jax.lax.broadcasted_iota(jnp.int32, (s, W), 1)
                  == c_ref[...]).astype(jnp.float32)
        pos_ref[...] = (
            jax.lax.dot(row_oh, row_tab_ref[...],
                        preferred_element_type=jnp.float32)
            + jax.lax.dot(col_oh, col_tab_ref[...],
                          preferred_element_type=jnp.float32))

    s = pos_ref.shape[0]
    for i in range(out_ref.shape[0] // s):
        out_ref[i * s:(i + 1) * s, :] = (
            out_in_ref[i * s:(i + 1) * s, :] + pos_ref[...])


def _tc2_kernel(pos_ref, out_in_ref, alias_ref, out_ref):
    del alias_ref
    s = pos_ref.shape[0]
    for i in range(out_ref.shape[0] // s):
        out_ref[i * s:(i + 1) * s, :] = (
            out_in_ref[i * s:(i + 1) * s, :] + pos_ref[...])


def kernel(output, row_table, col_table, r, c):
    B, S, D = output.shape
    flat = output.reshape(B * S, D)
    nb = _BATCHES_PER_BLOCK
    rows = nb * S
    b1 = _TC1_BATCHES

    pos_sc = _gather_pos_sc(row_table, col_table, r, c)

    r2 = r.reshape(S, 1)
    c2 = c.reshape(S, 1)
    out1 = pl.pallas_call(
        _tc1_kernel,
        grid=(b1 // nb,),
        in_specs=[
            pl.BlockSpec((S, 1), lambda b: (0, 0)),
            pl.BlockSpec((S, 1), lambda b: (0, 0)),
            pl.BlockSpec((H, D), lambda b: (0, 0)),
            pl.BlockSpec((W, D), lambda b: (0, 0)),
            pl.BlockSpec((rows, D), lambda b: (b, 0)),
        ],
        out_specs=pl.BlockSpec((rows, D), lambda b: (b, 0)),
        out_shape=jax.ShapeDtypeStruct((B * S, D), jnp.float32),
        scratch_shapes=[pltpu.VMEM((S, D), jnp.float32)],
    )(r2, c2, row_table, col_table, flat)

    off = b1 // nb
    res = pl.pallas_call(
        _tc2_kernel,
        grid=((B - b1) // nb,),
        in_specs=[
            pl.BlockSpec((S, D), lambda b: (0, 0)),
            pl.BlockSpec((rows, D), lambda b: (b + off, 0)),
            pl.BlockSpec((8, 128), lambda b: (0, 0)),
        ],
        out_specs=pl.BlockSpec((rows, D), lambda b: (b + off, 0)),
        out_shape=jax.ShapeDtypeStruct((B * S, D), jnp.float32),
        input_output_aliases={2: 0},
    )(pos_sc, flat, out1)
    return res.reshape(B, S, D)


# speedup vs baseline: 1.3588x; 1.2978x over previous
import jax
import jax.numpy as jnp
from jax.experimental import pallas as pl
from jax.experimental.pallas import tpu as pltpu

H = 32
W = 32


def _add_pos_kernel(r_ref, c_ref, row_tab_ref, col_tab_ref, out_in_ref,
                    out_ref, pos_ref):
    b = pl.program_id(0)

    @pl.when(b == 0)
    def _():
        s = r_ref.shape[0]
        row_oh = (jax.lax.broadcasted_iota(jnp.int32, (s, H), 1)
                  == r_ref[...]).astype(jnp.float32)
        col_oh = (jax.lax.broadcasted_iota(jnp.int32, (s, W), 1)
                  == c_ref[...]).astype(jnp.float32)
        pos_ref[...] = (
            jax.lax.dot(row_oh, row_tab_ref[...],
                        preferred_element_type=jnp.float32)
            + jax.lax.dot(col_oh, col_tab_ref[...],
                          preferred_element_type=jnp.float32))

    s = pos_ref.shape[0]
    nrep = out_ref.shape[0] // s
    for i in range(nrep):
        out_ref[i * s:(i + 1) * s, :] = (
            out_in_ref[i * s:(i + 1) * s, :] + pos_ref[...])


_BATCHES_PER_BLOCK = 8


def kernel(output, row_table, col_table, r, c):
    B, S, D = output.shape
    r2 = r.reshape(S, 1)
    c2 = c.reshape(S, 1)
    flat = output.reshape(B * S, D)
    nb = _BATCHES_PER_BLOCK
    rows = nb * S
    res = pl.pallas_call(
        _add_pos_kernel,
        grid=(B // nb,),
        in_specs=[
            pl.BlockSpec((S, 1), lambda b: (0, 0)),
            pl.BlockSpec((S, 1), lambda b: (0, 0)),
            pl.BlockSpec((H, D), lambda b: (0, 0)),
            pl.BlockSpec((W, D), lambda b: (0, 0)),
            pl.BlockSpec((rows, D), lambda b: (b, 0)),
        ],
        out_specs=pl.BlockSpec((rows, D), lambda b: (b, 0)),
        out_shape=jax.ShapeDtypeStruct((B * S, D), jnp.float32),
        scratch_shapes=[pltpu.VMEM((S, D), jnp.float32)],
    )(r2, c2, row_table, col_table, flat)
    return res.reshape(B, S, D)
